# Initial kernel scaffold; baseline (speedup 1.0000x reference)
import jax
import jax.numpy as jnp
from jax import lax
from jax.experimental import pallas as pl
from jax.experimental.pallas import tpu as pltpu
from jax.experimental.pallas import tpu_sc as plsc

_N = 10000
_D = 128
_B = 64
_E = 320000

_NCORE = 2
_NSUB = 16
_NW = _NCORE * _NSUB
_CH = 128
_STEPS = 80
_EPT = _CH * _STEPS
_EPAD = _EPT * _NW
_NPAD = 10240
_ZROWS = 64


def _sc_agg(src_hbm, dst_hbm, x_hbm, out_hbm, sidx, didx, rows, zbuf, agg_sh, sem):
    c = lax.axis_index("c")
    s = lax.axis_index("s")
    wid = c * _NSUB + s

    zeros16 = jnp.zeros((16,), jnp.float32)

    def _zb(r, carry):
        for j in range(_D // 16):
            zbuf[r, pl.ds(j * 16, 16)] = zeros16
        return carry

    lax.fori_loop(0, _ZROWS, _zb, 0)

    rows_per_sub = _NPAD // _NSUB

    def _zs(i, carry):
        pltpu.sync_copy(zbuf, agg_sh.at[pl.ds(s * rows_per_sub + i * _ZROWS, _ZROWS)])
        return carry

    lax.fori_loop(0, rows_per_sub // _ZROWS, _zs, 0)
    plsc.subcore_barrier()

    base = wid * _EPT

    def _step(k, carry):
        off = base + k * _CH
        pltpu.sync_copy(src_hbm.at[pl.ds(off, _CH)], sidx)
        pltpu.sync_copy(dst_hbm.at[pl.ds(off, _CH)], didx)
        pltpu.async_copy(x_hbm.at[sidx], rows, sem).wait()
        pltpu.sync_copy(rows, agg_sh.at[didx], add=True)
        return carry

    lax.fori_loop(0, _STEPS, _step, 0)
    plsc.subcore_barrier()

    wrows = _N // _NSUB
    pltpu.sync_copy(agg_sh.at[pl.ds(s * wrows, wrows)],
                    out_hbm.at[c, pl.ds(s * wrows, wrows)])


def _tc_dense(x_ref, a_ref, bcol_ref, brow_ref, W1_ref, b1_ref, g_ref, bb_ref,
              W2_ref, b2_ref, l1W_ref, l1b_ref, l2W_ref, l2b_ref,
              sig_ref, out_ref):
    h = x_ref[...] + a_ref[0] + a_ref[1]
    h = jnp.dot(h, W1_ref[...], preferred_element_type=jnp.float32) + b1_ref[...]
    mean = jnp.mean(h, axis=0)
    hcen = h - mean
    var = jnp.mean(hcen * hcen, axis=0)
    h = hcen * (g_ref[...] * lax.rsqrt(var + 1e-5)) + bb_ref[...]
    h = jnp.maximum(h, 0.0)
    h = jnp.dot(h, W2_ref[...], preferred_element_type=jnp.float32) + b2_ref[...]
    h = jnp.maximum(h, 0.0)

    bcol = bcol_ref[...]
    segs = lax.broadcasted_iota(jnp.int32, (_B, _N), 0)
    P = (brow_ref[...] == segs).astype(jnp.float32)
    h_add = jnp.dot(P, h, preferred_element_type=jnp.float32)

    rowid = lax.broadcasted_iota(jnp.int32, (_B, 1), 0)

    def _mbody(b, acc):
        m = jnp.max(jnp.where(bcol == b, h, -jnp.inf), axis=0)
        return jnp.where(rowid == b, m.reshape(1, _D), acc)

    h_max = lax.fori_loop(0, _B, _mbody,
                          jnp.full((_B, _D), -jnp.inf, jnp.float32))

    hc = jnp.concatenate([h_add, h_max], axis=1)
    hc = jnp.dot(hc, l1W_ref[...], preferred_element_type=jnp.float32) + l1b_ref[...]
    hc = jnp.maximum(hc, 0.0)
    o = jnp.dot(hc, l2W_ref[...], preferred_element_type=jnp.float32) + l2b_ref[...]
    out_ref[...] = o
    sig_ref[...] = 1.0 / (1.0 + jnp.exp(-o))


def kernel(x, edge_index, batch, W1, b1, bn_g, bn_b, W2, b2, l1W, l1b, l2W, l2b):
    src = edge_index[0]
    dst = edge_index[1]
    pad = _EPAD - _E
    src_p = jnp.concatenate([src, jnp.zeros((pad,), jnp.int32)])
    dst_p = jnp.concatenate(
        [dst, _N + (jnp.arange(pad, dtype=jnp.int32) % (_NPAD - _N))])

    agg2 = pl.kernel(
        _sc_agg,
        out_type=jax.ShapeDtypeStruct((_NCORE, _N, _D), jnp.float32),
        mesh=plsc.VectorSubcoreMesh(core_axis_name="c", subcore_axis_name="s"),
        scratch_types=[
            pltpu.VMEM((_CH,), jnp.int32),
            pltpu.VMEM((_CH,), jnp.int32),
            pltpu.VMEM((_CH, _D), jnp.float32),
            pltpu.VMEM((_ZROWS, _D), jnp.float32),
            pltpu.VMEM_SHARED((_NPAD, _D), jnp.float32),
            pltpu.SemaphoreType.DMA,
        ],
    )(src_p, dst_p, x)

    bcol = batch.astype(jnp.int32).reshape(_N, 1)
    brow = batch.astype(jnp.int32).reshape(1, _N)

    sig, out = pl.pallas_call(
        _tc_dense,
        out_shape=[
            jax.ShapeDtypeStruct((_B, 1), jnp.float32),
            jax.ShapeDtypeStruct((_B, 1), jnp.float32),
        ],
    )(x, agg2, bcol, brow,
      W1, b1.reshape(1, _D), bn_g.reshape(1, _D), bn_b.reshape(1, _D),
      W2, b2.reshape(1, _D), l1W, l1b.reshape(1, 2 * _D),
      l2W, l2b.reshape(1, 1))
    return (sig, out)


# speedup vs baseline: 2.5798x; 2.5798x over previous
import jax
import jax.numpy as jnp
from jax import lax
from jax.experimental import pallas as pl
from jax.experimental.pallas import tpu as pltpu
from jax.experimental.pallas import tpu_sc as plsc

_N = 10000
_D = 128
_B = 64
_E = 320000

_NCORE = 2
_NSUB = 16
_NW = _NCORE * _NSUB
_CH = 128
_STEPS = 80
_EPT = _CH * _STEPS
_EPAD = _EPT * _NW
_NPAD = 10240
_ZROWS = 64


def _sc_agg(src_hbm, dst_hbm, x_hbm, out_hbm, sidx, didx, rows, zbuf, agg_sh, sem):
    c = lax.axis_index("c")
    s = lax.axis_index("s")
    wid = c * _NSUB + s

    zeros16 = jnp.zeros((16,), jnp.float32)

    def _zb(r, carry):
        for j in range(_D // 16):
            zbuf[r, pl.ds(j * 16, 16)] = zeros16
        return carry

    lax.fori_loop(0, _ZROWS, _zb, 0)

    rows_per_sub = _NPAD // _NSUB

    def _zs(i, carry):
        pltpu.sync_copy(zbuf, agg_sh.at[pl.ds(s * rows_per_sub + i * _ZROWS, _ZROWS)])
        return carry

    lax.fori_loop(0, rows_per_sub // _ZROWS, _zs, 0)
    plsc.subcore_barrier()

    base = wid * _EPT

    def _step(k, carry):
        off = base + k * _CH
        pltpu.sync_copy(src_hbm.at[pl.ds(off, _CH)], sidx)
        pltpu.sync_copy(dst_hbm.at[pl.ds(off, _CH)], didx)
        pltpu.async_copy(x_hbm.at[sidx], rows, sem).wait()
        pltpu.sync_copy(rows, agg_sh.at[didx], add=True)
        return carry

    lax.fori_loop(0, _STEPS, _step, 0)
    plsc.subcore_barrier()

    wrows = 624
    pltpu.sync_copy(agg_sh.at[pl.ds(s * wrows, wrows)],
                    out_hbm.at[c, pl.ds(s * wrows, wrows)])

    tail = _N - wrows * _NSUB

    @pl.when(s == 0)
    def _wb_tail():
        pltpu.sync_copy(agg_sh.at[pl.ds(wrows * _NSUB, tail)],
                        out_hbm.at[c, pl.ds(wrows * _NSUB, tail)])


def _tc_dense(x_ref, a_ref, bcol_ref, brow_ref, W1_ref, b1_ref, g_ref, bb_ref,
              W2_ref, b2_ref, l1W_ref, l1b_ref, l2W_ref, l2b_ref,
              sig_ref, out_ref):
    h = x_ref[...] + a_ref[0] + a_ref[1]
    h = jnp.dot(h, W1_ref[...], preferred_element_type=jnp.float32, precision=lax.Precision.HIGHEST) + b1_ref[...]
    mean = jnp.mean(h, axis=0)
    hcen = h - mean
    var = jnp.mean(hcen * hcen, axis=0)
    h = hcen * (g_ref[...] * lax.rsqrt(var + 1e-5)) + bb_ref[...]
    h = jnp.maximum(h, 0.0)
    h = jnp.dot(h, W2_ref[...], preferred_element_type=jnp.float32, precision=lax.Precision.HIGHEST) + b2_ref[...]
    h = jnp.maximum(h, 0.0)

    bcol = bcol_ref[...]
    segs = lax.broadcasted_iota(jnp.int32, (_B, _N), 0)
    P = (brow_ref[...] == segs).astype(jnp.float32)
    h_add = jnp.dot(P, h, preferred_element_type=jnp.float32, precision=lax.Precision.HIGHEST)

    rowid = lax.broadcasted_iota(jnp.int32, (_B, 1), 0)

    def _mbody(b, acc):
        m = jnp.max(jnp.where(bcol == b, h, -jnp.inf), axis=0)
        return jnp.where(rowid == b, m.reshape(1, _D), acc)

    h_max = lax.fori_loop(0, _B, _mbody,
                          jnp.full((_B, _D), -jnp.inf, jnp.float32))

    hc = jnp.concatenate([h_add, h_max], axis=1)
    hc = jnp.dot(hc, l1W_ref[...], preferred_element_type=jnp.float32, precision=lax.Precision.HIGHEST) + l1b_ref[...]
    hc = jnp.maximum(hc, 0.0)
    o = jnp.dot(hc, l2W_ref[...], preferred_element_type=jnp.float32, precision=lax.Precision.HIGHEST) + l2b_ref[...]
    out_ref[...] = o
    sig_ref[...] = 1.0 / (1.0 + jnp.exp(-o))


def kernel(x, edge_index, batch, W1, b1, bn_g, bn_b, W2, b2, l1W, l1b, l2W, l2b):
    src = edge_index[0]
    dst = edge_index[1]
    pad = _EPAD - _E
    src_p = jnp.concatenate([src, jnp.zeros((pad,), jnp.int32)])
    dst_p = jnp.concatenate(
        [dst, _N + (jnp.arange(pad, dtype=jnp.int32) % (_NPAD - _N))])

    agg2 = pl.kernel(
        _sc_agg,
        out_type=jax.ShapeDtypeStruct((_NCORE, _N, _D), jnp.float32),
        mesh=plsc.VectorSubcoreMesh(core_axis_name="c", subcore_axis_name="s"),
        scratch_types=[
            pltpu.VMEM((_CH,), jnp.int32),
            pltpu.VMEM((_CH,), jnp.int32),
            pltpu.VMEM((_CH, _D), jnp.float32),
            pltpu.VMEM((_ZROWS, _D), jnp.float32),
            pltpu.VMEM_SHARED((_NPAD, _D), jnp.float32),
            pltpu.SemaphoreType.DMA,
        ],
    )(src_p, dst_p, x)

    bcol = batch.astype(jnp.int32).reshape(_N, 1)
    brow = batch.astype(jnp.int32).reshape(1, _N)

    sig, out = pl.pallas_call(
        _tc_dense,
        out_shape=[
            jax.ShapeDtypeStruct((_B, 1), jnp.float32),
            jax.ShapeDtypeStruct((_B, 1), jnp.float32),
        ],
    )(x, agg2, bcol, brow,
      W1, b1.reshape(1, _D), bn_g.reshape(1, _D), bn_b.reshape(1, _D),
      W2, b2.reshape(1, _D), l1W, l1b.reshape(1, 2 * _D),
      l2W, l2b.reshape(1, 1))
    return (sig, out)


# speedup vs baseline: 3.1346x; 1.2151x over previous
import jax
import jax.numpy as jnp
from jax import lax
from jax.experimental import pallas as pl
from jax.experimental.pallas import tpu as pltpu
from jax.experimental.pallas import tpu_sc as plsc

_N = 10000
_D = 128
_B = 64
_E = 320000

_NCORE = 2
_NSUB = 16
_NW = _NCORE * _NSUB
_CH = 128
_STEPS = 80
_EPT = _CH * _STEPS
_EPAD = _EPT * _NW
_NPAD = 10240
_ZROWS = 64


def _sc_agg(src_hbm, dst_hbm, x_hbm, out_hbm,
            sidx0, sidx1, didx0, didx1, rows0, rows1, zbuf, agg_sh,
            ssem0, ssem1, dsem0, dsem1, gsem0, gsem1):
    c = lax.axis_index("c")
    s = lax.axis_index("s")
    wid = c * _NSUB + s

    zeros16 = jnp.zeros((16,), jnp.float32)

    def _zb(r, carry):
        for j in range(_D // 16):
            zbuf[r, pl.ds(j * 16, 16)] = zeros16
        return carry

    lax.fori_loop(0, _ZROWS, _zb, 0)

    rows_per_sub = _NPAD // _NSUB

    def _zs(i, carry):
        pltpu.sync_copy(zbuf, agg_sh.at[pl.ds(s * rows_per_sub + i * _ZROWS, _ZROWS)])
        return carry

    lax.fori_loop(0, rows_per_sub // _ZROWS, _zs, 0)
    plsc.subcore_barrier()

    sidx_l = (sidx0, sidx1)
    didx_l = (didx0, didx1)
    rows_l = (rows0, rows1)
    ssem_l = (ssem0, ssem1)
    dsem_l = (dsem0, dsem1)
    gsem_l = (gsem0, gsem1)
    base = wid * _EPT

    def _sload(k, b):
        return pltpu.make_async_copy(src_hbm.at[pl.ds(base + k * _CH, _CH)],
                                     sidx_l[b], ssem_l[b])

    def _dload(k, b):
        return pltpu.make_async_copy(dst_hbm.at[pl.ds(base + k * _CH, _CH)],
                                     didx_l[b], dsem_l[b])

    def _gather(b):
        return pltpu.make_async_copy(x_hbm.at[sidx_l[b]], rows_l[b], gsem_l[b])

    def _scatter(b):
        pltpu.sync_copy(rows_l[b], agg_sh.at[didx_l[b]], add=True)

    _sload(0, 0).start()
    _dload(0, 0).start()
    _sload(1, 1).start()
    _dload(1, 1).start()
    _sload(0, 0).wait()
    _gather(0).start()

    @pl.loop(0, _STEPS - 2, step=2)
    def _steady(g):
        for b in range(2):
            k = g + b
            nb = 1 - b
            _sload(k + 1, nb).wait()
            _gather(nb).start()
            _gather(b).wait()
            _dload(k, b).wait()
            _scatter(b)
            _sload(k + 2, b).start()
            _dload(k + 2, b).start()

    _sload(_STEPS - 1, 1).wait()
    _gather(1).start()
    _gather(0).wait()
    _dload(_STEPS - 2, 0).wait()
    _scatter(0)
    _gather(1).wait()
    _dload(_STEPS - 1, 1).wait()
    _scatter(1)
    plsc.subcore_barrier()

    wrows = 624
    pltpu.sync_copy(agg_sh.at[pl.ds(s * wrows, wrows)],
                    out_hbm.at[c, pl.ds(s * wrows, wrows)])

    tail = _N - wrows * _NSUB

    @pl.when(s == 0)
    def _wb_tail():
        pltpu.sync_copy(agg_sh.at[pl.ds(wrows * _NSUB, tail)],
                        out_hbm.at[c, pl.ds(wrows * _NSUB, tail)])


def _tc_dense(x_ref, a_ref, bcol_ref, brow_ref, W1_ref, b1_ref, g_ref, bb_ref,
              W2_ref, b2_ref, l1W_ref, l1b_ref, l2W_ref, l2b_ref,
              sig_ref, out_ref):
    h = x_ref[...] + a_ref[0] + a_ref[1]
    h = jnp.dot(h, W1_ref[...], preferred_element_type=jnp.float32) + b1_ref[...]
    mean = jnp.mean(h, axis=0)
    hcen = h - mean
    var = jnp.mean(hcen * hcen, axis=0)
    h = hcen * (g_ref[...] * lax.rsqrt(var + 1e-5)) + bb_ref[...]
    h = jnp.maximum(h, 0.0)
    h = jnp.dot(h, W2_ref[...], preferred_element_type=jnp.float32) + b2_ref[...]
    h = jnp.maximum(h, 0.0)

    bcol = bcol_ref[...]
    segs = lax.broadcasted_iota(jnp.int32, (_B, _N), 0)
    P = (brow_ref[...] == segs).astype(jnp.float32)
    h_add = jnp.dot(P, h, preferred_element_type=jnp.float32, precision=lax.Precision.HIGHEST)

    rowid = lax.broadcasted_iota(jnp.int32, (_B, 1), 0)

    def _mbody(b, acc):
        m = jnp.max(jnp.where(bcol == b, h, -jnp.inf), axis=0)
        return jnp.where(rowid == b, m.reshape(1, _D), acc)

    h_max = lax.fori_loop(0, _B, _mbody,
                          jnp.full((_B, _D), -jnp.inf, jnp.float32))

    hc = jnp.concatenate([h_add, h_max], axis=1)
    hc = jnp.dot(hc, l1W_ref[...], preferred_element_type=jnp.float32) + l1b_ref[...]
    hc = jnp.maximum(hc, 0.0)
    o = jnp.dot(hc, l2W_ref[...], preferred_element_type=jnp.float32) + l2b_ref[...]
    out_ref[...] = o
    sig_ref[...] = 1.0 / (1.0 + jnp.exp(-o))


def _run_sc_agg(x, src, dst):
    pad = _EPAD - _E
    src_p = jnp.concatenate([src, jnp.zeros((pad,), jnp.int32)])
    dst_p = jnp.concatenate(
        [dst, _N + (jnp.arange(pad, dtype=jnp.int32) % (_NPAD - _N))])
    return pl.kernel(
        _sc_agg,
        out_type=jax.ShapeDtypeStruct((_NCORE, _N, _D), jnp.float32),
        mesh=plsc.VectorSubcoreMesh(core_axis_name="c", subcore_axis_name="s"),
        scratch_types=[
            pltpu.VMEM((_CH,), jnp.int32),
            pltpu.VMEM((_CH,), jnp.int32),
            pltpu.VMEM((_CH,), jnp.int32),
            pltpu.VMEM((_CH,), jnp.int32),
            pltpu.VMEM((_CH, _D), jnp.float32),
            pltpu.VMEM((_CH, _D), jnp.float32),
            pltpu.VMEM((_ZROWS, _D), jnp.float32),
            pltpu.VMEM_SHARED((_NPAD, _D), jnp.float32),
            pltpu.SemaphoreType.DMA,
            pltpu.SemaphoreType.DMA,
            pltpu.SemaphoreType.DMA,
            pltpu.SemaphoreType.DMA,
            pltpu.SemaphoreType.DMA,
            pltpu.SemaphoreType.DMA,
        ],
    )(src_p, dst_p, x)


def kernel(x, edge_index, batch, W1, b1, bn_g, bn_b, W2, b2, l1W, l1b, l2W, l2b):
    agg2 = _run_sc_agg(x, edge_index[0], edge_index[1])

    bcol = batch.astype(jnp.int32).reshape(_N, 1)
    brow = batch.astype(jnp.int32).reshape(1, _N)

    sig, out = pl.pallas_call(
        _tc_dense,
        out_shape=[
            jax.ShapeDtypeStruct((_B, 1), jnp.float32),
            jax.ShapeDtypeStruct((_B, 1), jnp.float32),
        ],
    )(x, agg2, bcol, brow,
      W1, b1.reshape(1, _D), bn_g.reshape(1, _D), bn_b.reshape(1, _D),
      W2, b2.reshape(1, _D), l1W, l1b.reshape(1, 2 * _D),
      l2W, l2b.reshape(1, 1))
    return (sig, out)


# speedup vs baseline: 4.0071x; 1.2783x over previous
import jax
import jax.numpy as jnp
from jax import lax
from jax.experimental import pallas as pl
from jax.experimental.pallas import tpu as pltpu
from jax.experimental.pallas import tpu_sc as plsc

_N = 10000
_D = 128
_DH = 64
_B = 64
_E = 320000

_NCORE = 2
_NSUB = 16
_CH = 128
_GCH = 16
_NGRP = 10
_CPT = _GCH * _NGRP
_EPT = _CH * _CPT
_ROWS2D = 2592
_NPAD = 10240


def _sc_agg(src_hbm, dst_hbm, xs_hbm, out_hbm,
            sg0, sg1, dg0, dg1, dw, r0, r1, r2, r3, r4, r5, r6, r7,
            zbuf, agg_sh,
            sgs0, sgs1, dgs0, dgs1, g0, g1, g2, g3, g4, g5, g6, g7):
    c = lax.axis_index("c")
    s = lax.axis_index("s")

    zeros16 = jnp.zeros((16,), jnp.float32)

    def _zb(r, carry):
        for jj in range(_DH // 16):
            zbuf[r, pl.ds(jj * 16, 16)] = zeros16
        return carry

    lax.fori_loop(0, 128, _zb, 0)

    gsem = (g0, g1, g2, g3, g4, g5, g6, g7)
    rows = (r0, r1, r2, r3, r4, r5, r6, r7)
    sg = (sg0, sg1)
    dg = (dg0, dg1)
    sgs = (sgs0, sgs1)
    dgs = (dgs0, dgs1)

    for i in range(5):
        pltpu.make_async_copy(zbuf, agg_sh.at[pl.ds(s * 640 + i * 128, 128)],
                              gsem[i]).start()
    for i in range(5):
        pltpu.make_async_copy(zbuf, agg_sh.at[pl.ds(s * 640 + i * 128, 128)],
                              gsem[i]).wait()
    plsc.subcore_barrier()

    rbase = s * _CPT

    def _sload(G, p):
        return pltpu.make_async_copy(src_hbm.at[pl.ds(rbase + G * _GCH, _GCH)],
                                     sg[p], sgs[p])

    def _dload(G, p):
        return pltpu.make_async_copy(dst_hbm.at[pl.ds(rbase + G * _GCH, _GCH)],
                                     dg[p], dgs[p])

    def _gather(p, j, rb):
        return pltpu.make_async_copy(xs_hbm.at[c].at[sg[p].at[j]],
                                     rows[rb], gsem[rb])

    def _scatter(p, j, rb):
        for jj in range(_CH // 16):
            dw[pl.ds(jj * 16, 16)] = dg[p][j, pl.ds(jj * 16, 16)]
        pltpu.sync_copy(rows[rb], agg_sh.at[dw], add=True)

    _sload(0, 0).start()
    _dload(0, 0).start()
    _sload(1, 1).start()
    _dload(1, 1).start()
    _sload(0, 0).wait()
    _dload(0, 0).wait()
    for j in range(7):
        _gather(0, j, j).start()

    @pl.loop(0, _NGRP, step=2)
    def _grp(g):
        for gg in range(2):
            p = gg
            for j in range(_GCH):
                rb = j % 8
                _gather(p, j, rb).wait()
                if j == 8:
                    _sload(g + gg + 1, 1 - p).wait()
                    _dload(g + gg + 1, 1 - p).wait()
                _scatter(p, j, rb)
                if j < 9:
                    _gather(p, j + 7, (j + 7) % 8).start()
                else:
                    _gather(1 - p, j - 9, (j + 7) % 8).start()
                if j == 15:
                    _sload(g + gg + 2, p).start()
                    _dload(g + gg + 2, p).start()

    for j in range(7):
        _gather(0, j, j).wait()
    _sload(_NGRP + 1, 1).wait()
    _dload(_NGRP + 1, 1).wait()
    plsc.subcore_barrier()

    wrows = 624
    pltpu.sync_copy(agg_sh.at[pl.ds(s * wrows, wrows)],
                    out_hbm.at[c, pl.ds(s * wrows, wrows)])

    tail = _N - wrows * _NSUB

    @pl.when(s == 0)
    def _wb_tail():
        pltpu.sync_copy(agg_sh.at[pl.ds(wrows * _NSUB, tail)],
                        out_hbm.at[c, pl.ds(wrows * _NSUB, tail)])


def _tc_dense(x_ref, a_ref, bcol_ref, brow_ref, W1_ref, b1_ref, g_ref, bb_ref,
              W2_ref, b2_ref, l1W_ref, l1b_ref, l2W_ref, l2b_ref,
              sig_ref, out_ref):
    h = x_ref[...] + jnp.concatenate([a_ref[0], a_ref[1]], axis=1)
    h = jnp.dot(h, W1_ref[...], preferred_element_type=jnp.float32) + b1_ref[...]
    mean = jnp.mean(h, axis=0)
    hcen = h - mean
    var = jnp.mean(hcen * hcen, axis=0)
    h = hcen * (g_ref[...] * lax.rsqrt(var + 1e-5)) + bb_ref[...]
    h = jnp.maximum(h, 0.0)
    h = jnp.dot(h, W2_ref[...], preferred_element_type=jnp.float32) + b2_ref[...]
    h = jnp.maximum(h, 0.0)

    bcol = bcol_ref[...]
    segs = lax.broadcasted_iota(jnp.int32, (_B, _N), 0)
    P = (brow_ref[...] == segs).astype(jnp.float32)
    h_add = jnp.dot(P, h, preferred_element_type=jnp.float32,
                    precision=lax.Precision.HIGHEST)

    rowid = lax.broadcasted_iota(jnp.int32, (_B, 1), 0)

    def _mbody(b, acc):
        m = jnp.max(jnp.where(bcol == b, h, -jnp.inf), axis=0)
        return jnp.where(rowid == b, m.reshape(1, _D), acc)

    h_max = lax.fori_loop(0, _B, _mbody,
                          jnp.full((_B, _D), -jnp.inf, jnp.float32))

    hc = jnp.concatenate([h_add, h_max], axis=1)
    hc = jnp.dot(hc, l1W_ref[...], preferred_element_type=jnp.float32) + l1b_ref[...]
    hc = jnp.maximum(hc, 0.0)
    o = jnp.dot(hc, l2W_ref[...], preferred_element_type=jnp.float32) + l2b_ref[...]
    out_ref[...] = o
    sig_ref[...] = 1.0 / (1.0 + jnp.exp(-o))


def _run_sc_agg(x, src, dst):
    pad = _ROWS2D * _CH - _E
    src2 = jnp.concatenate([src, jnp.zeros((pad,), jnp.int32)]).reshape(_ROWS2D, _CH)
    dst2 = jnp.concatenate(
        [dst, _N + (jnp.arange(pad, dtype=jnp.int32) % (_NPAD - _N))]
    ).reshape(_ROWS2D, _CH)
    xs = jnp.stack([x[:, :_DH], x[:, _DH:]])

    return pl.kernel(
        _sc_agg,
        out_type=jax.ShapeDtypeStruct((_NCORE, _N, _DH), jnp.float32),
        mesh=plsc.VectorSubcoreMesh(core_axis_name="c", subcore_axis_name="s"),
        compiler_params=pltpu.CompilerParams(use_tc_tiling_on_sc=False),
        scratch_types=[
            pltpu.VMEM((_GCH, _CH), jnp.int32),
            pltpu.VMEM((_GCH, _CH), jnp.int32),
            pltpu.VMEM((_GCH, _CH), jnp.int32),
            pltpu.VMEM((_GCH, _CH), jnp.int32),
            pltpu.VMEM((_CH,), jnp.int32),
        ] + [pltpu.VMEM((_CH, _DH), jnp.float32)] * 8 + [
            pltpu.VMEM((128, _DH), jnp.float32),
            pltpu.VMEM_SHARED((_NPAD, _DH), jnp.float32),
        ] + [pltpu.SemaphoreType.DMA] * 12,
    )(src2, dst2, xs)


def kernel(x, edge_index, batch, W1, b1, bn_g, bn_b, W2, b2, l1W, l1b, l2W, l2b):
    agg2 = _run_sc_agg(x, edge_index[0], edge_index[1])

    bcol = batch.astype(jnp.int32).reshape(_N, 1)
    brow = batch.astype(jnp.int32).reshape(1, _N)

    sig, out = pl.pallas_call(
        _tc_dense,
        out_shape=[
            jax.ShapeDtypeStruct((_B, 1), jnp.float32),
            jax.ShapeDtypeStruct((_B, 1), jnp.float32),
        ],
    )(x, agg2, bcol, brow,
      W1, b1.reshape(1, _D), bn_g.reshape(1, _D), bn_b.reshape(1, _D),
      W2, b2.reshape(1, _D), l1W, l1b.reshape(1, 2 * _D),
      l2W, l2b.reshape(1, 1))
    return (sig, out)
